# Initial kernel scaffold; baseline (speedup 1.0000x reference)
import jax
import jax.numpy as jnp
from jax.experimental import pallas as pl


def kernel(x, table, W1, W2, W3):
    raise NotImplementedError("write your pallas kernel here")


# speedup vs baseline: 303.6939x; 303.6939x over previous
import functools

import numpy as np
import jax
import jax.numpy as jnp
from jax import lax
from jax.experimental import pallas as pl
from jax.experimental.pallas import tpu as pltpu
from jax.experimental.pallas import tpu_sc as plsc

_L = 16
_F = 2
_T = 16384
_BASE_RES = 16
_SCALE = 1.4472692012786865
_RES = [int(np.floor(_BASE_RES * (_SCALE ** l))) for l in range(_L)]
_P2 = -1640531535
_P3 = 805459861
_CORNERS = ((0, 0, 0), (0, 0, 1), (0, 1, 0), (0, 1, 1),
            (1, 0, 0), (1, 0, 1), (1, 1, 0), (1, 1, 1))
_MIN_ROUGH, _MAX_ROUGH = 0.08, 1.0

_LANES = 16


def _sc_encode(x_t, tbl0, tbl1, *, interpret=False):
    n = x_t.shape[1]
    nc, ns = 2, 16
    nw = nc * ns
    pt = n // nw
    assert pt * nw == n and pt % _LANES == 0
    c = min(pt, 16384)
    nchunk = pt // c
    assert nchunk * c == pt

    mesh = plsc.VectorSubcoreMesh(core_axis_name="c", subcore_axis_name="s",
                                  num_cores=nc, num_subcores=ns)

    @functools.partial(
        pl.kernel,
        out_type=jax.ShapeDtypeStruct((2 * _L, n), jnp.float32),
        mesh=mesh,
        scratch_types=[
            pltpu.VMEM((3, c), jnp.float32),
            pltpu.VMEM((_T,), jnp.float32),
            pltpu.VMEM((_T,), jnp.float32),
            pltpu.VMEM((2, c), jnp.float32),
        ],
        compiler_params=pltpu.CompilerParams(needs_layout_passes=False),
        interpret=interpret,
    )
    def enc_kernel(x_hbm, t0_hbm, t1_hbm, out_hbm, xbuf, tb0, tb1, stag):
        wid = lax.axis_index("s") * nc + lax.axis_index("c")
        base = wid * pt

        def chunk_body(k, carry):
            cb = base + k * c
            pltpu.sync_copy(x_hbm.at[:, pl.ds(cb, c)], xbuf)
            for l in range(_L):
                res = _RES[l]
                stride = res + 2
                dense = stride ** 3 <= _T
                pltpu.sync_copy(t0_hbm.at[pl.ds(l * _T, _T)], tb0)
                pltpu.sync_copy(t1_hbm.at[pl.ds(l * _T, _T)], tb1)

                def vbody(i, cr, res=res, stride=stride, dense=dense):
                    s = pl.ds(i * _LANES, _LANES)
                    xv = xbuf[0, s]
                    yv = xbuf[1, s]
                    zv = xbuf[2, s]
                    rf = jnp.float32(res)
                    px = xv * rf + 0.5
                    py = yv * rf + 0.5
                    pz = zv * rf + 0.5
                    cx = px.astype(jnp.int32)
                    cy = py.astype(jnp.int32)
                    cz = pz.astype(jnp.int32)
                    wx = px - cx.astype(jnp.float32)
                    wy = py - cy.astype(jnp.float32)
                    wz = pz - cz.astype(jnp.float32)
                    if dense:
                        s2 = stride * stride
                        hy0 = cy * stride
                        hz0 = cz * s2
                        hx = (cx, cx + 1)
                        hy = (hy0, hy0 + stride)
                        hz = (hz0, hz0 + s2)

                        def cidx(a, b, cc):
                            return hx[a] + hy[b] + hz[cc]
                    else:
                        hy0 = cy * _P2
                        hz0 = cz * _P3
                        hx = (cx, cx + 1)
                        hy = (hy0, hy0 + _P2)
                        hz = (hz0, hz0 + _P3)

                        def cidx(a, b, cc):
                            return (hx[a] ^ hy[b] ^ hz[cc]) & (_T - 1)

                    ux = 1.0 - wx
                    uy = 1.0 - wy
                    uz = 1.0 - wz
                    wab = {(0, 0): ux * uy, (0, 1): ux * wy,
                           (1, 0): wx * uy, (1, 1): wx * wy}
                    f0 = jnp.zeros((_LANES,), jnp.float32)
                    f1 = jnp.zeros((_LANES,), jnp.float32)
                    for (a, b, cc) in _CORNERS:
                        idx = cidx(a, b, cc)
                        wt = wab[(a, b)] * (wz if cc else uz)
                        g0 = plsc.load_gather(tb0, [idx])
                        g1 = plsc.load_gather(tb1, [idx])
                        f0 = f0 + wt * g0
                        f1 = f1 + wt * g1
                    stag[0, s] = f0
                    stag[1, s] = f1
                    return cr

                lax.fori_loop(0, c // _LANES, vbody, 0)
                pltpu.sync_copy(stag, out_hbm.at[pl.ds(2 * l, 2), pl.ds(cb, c)])
            return carry

        lax.fori_loop(0, nchunk, chunk_body, 0)

    return enc_kernel(x_t, tbl0, tbl1)


def _mlp_body(x_ref, enc_ref, w1x_ref, w1e_ref, w2_ref, w3_ref,
              diff_ref, spec_ref, rough_ref):
    x2 = 2.0 * x_ref[...] - 1.0
    h = jnp.dot(w1e_ref[...], enc_ref[...], preferred_element_type=jnp.float32)
    h = h + jnp.dot(w1x_ref[...], x2, preferred_element_type=jnp.float32)
    h = jnp.maximum(h, 0.0)
    h = jnp.maximum(
        jnp.dot(w2_ref[...], h, preferred_element_type=jnp.float32), 0.0)
    o = jnp.dot(w3_ref[...], h, preferred_element_type=jnp.float32)
    diff_ref[...] = jax.nn.sigmoid(o[0:3])
    spec_ref[...] = 1.0 - jax.nn.sigmoid(o[3:4])
    r = jax.nn.sigmoid(o[4:5])
    rough_ref[...] = r * _MIN_ROUGH + (1.0 - r) * _MAX_ROUGH


def _mlp(x_t, enc, w1x, w1e, w2, w3, *, interpret=False):
    n = x_t.shape[1]
    b = min(n, 4096)
    grid = (n // b,)
    f32 = jnp.float32
    return pl.pallas_call(
        _mlp_body,
        grid=grid,
        in_specs=[
            pl.BlockSpec((3, b), lambda j: (0, j)),
            pl.BlockSpec((2 * _L, b), lambda j: (0, j)),
            pl.BlockSpec(w1x.shape, lambda j: (0, 0)),
            pl.BlockSpec(w1e.shape, lambda j: (0, 0)),
            pl.BlockSpec(w2.shape, lambda j: (0, 0)),
            pl.BlockSpec(w3.shape, lambda j: (0, 0)),
        ],
        out_specs=[
            pl.BlockSpec((3, b), lambda j: (0, j)),
            pl.BlockSpec((1, b), lambda j: (0, j)),
            pl.BlockSpec((1, b), lambda j: (0, j)),
        ],
        out_shape=[
            jax.ShapeDtypeStruct((3, n), f32),
            jax.ShapeDtypeStruct((1, n), f32),
            jax.ShapeDtypeStruct((1, n), f32),
        ],
        interpret=interpret,
    )(x_t, enc, w1x, w1e, w2, w3)


def kernel(x, table, W1, W2, W3):
    x_t = x.T
    tbl0 = table[:, :, 0].reshape(-1)
    tbl1 = table[:, :, 1].reshape(-1)
    enc = _sc_encode(x_t, tbl0, tbl1)
    diff_t, spec_t, rough_t = _mlp(
        x_t, enc, W1[:3].T, W1[3:].T, W2.T, W3.T)
    return diff_t.T, spec_t.T, rough_t.T
